# baseline (speedup 1.0000x reference)
import functools

import jax
import jax.numpy as jnp
from jax import lax
from jax.experimental import pallas as pl
from jax.experimental.pallas import tpu as pltpu
from jax.experimental.pallas import tpu_sc as plsc

_N = 10000
_E = 320000
_D = 128
_G = 64
_EPS = 1e-5

_NC = 2
_NS = 16
_NW = _NC * _NS
_CH = 128
_EPW = _E // _NW
_PAD = 240
_EPWP = _EPW + _PAD
_RPW = _EPWP // _CH
_NB = 16
_CPB = _RPW // _NB
_NT = 16
_NA = _N + _NT
_NPS = _NA // _NS

_CCH = 80
_CNB = 5
_CCPB = (_EPW // _CCH) // _CNB
_CNPS = _N // _NS


def _make_sc_agg():
    mesh = plsc.VectorSubcoreMesh(
        core_axis_name="c", subcore_axis_name="s",
        num_cores=_NC, num_subcores=_NS)
    out_type = jax.ShapeDtypeStruct((_NC, _NS, _NPS, _D), jnp.float32)
    scratch = [
        pltpu.VMEM((_CPB, _CH), jnp.int32),
        pltpu.VMEM((_CPB, _CH), jnp.int32),
        pltpu.VMEM((_CH, _D), jnp.float32),
        pltpu.VMEM((_CH, _D), jnp.float32),
        pltpu.VMEM_SHARED((_NA, _D), jnp.float32),
        pltpu.SemaphoreType.DMA,
        pltpu.SemaphoreType.DMA,
    ]

    def body(h_hbm, src_hbm, dst_hbm, zd_hbm, *refs):
        (out_hbm, idx_s, idx_d, rows0, rows1, agg_sh, sem0, sem1) = refs
        c = lax.axis_index("c")
        s = lax.axis_index("s")
        w = c * _NS + s

        pltpu.sync_copy(zd_hbm, agg_sh.at[pl.ds(s * _NPS, _NPS)])

        plsc.subcore_barrier()

        def blk(b, _):
            pltpu.sync_copy(src_hbm.at[w, b], idx_s)
            pltpu.sync_copy(dst_hbm.at[w, b], idx_d)

            pltpu.async_copy(h_hbm.at[idx_s.at[0]], rows0, sem0)

            def step(i, _):
                j = 2 * i
                pltpu.async_copy(h_hbm.at[idx_s.at[j + 1]], rows1, sem1)
                pltpu.make_async_copy(h_hbm.at[idx_s.at[j]], rows0,
                                      sem0).wait()
                pltpu.sync_copy(rows0, agg_sh.at[idx_d.at[j]], add=True)
                pltpu.async_copy(h_hbm.at[idx_s.at[j + 2]], rows0, sem0)
                pltpu.make_async_copy(h_hbm.at[idx_s.at[j + 1]], rows1,
                                      sem1).wait()
                pltpu.sync_copy(rows1, agg_sh.at[idx_d.at[j + 1]],
                                add=True)
                return 0
            lax.fori_loop(0, (_CPB - 1) // 2, step, 0)

            pltpu.make_async_copy(h_hbm.at[idx_s.at[_CPB - 1]], rows0,
                                  sem0).wait()
            pltpu.sync_copy(rows0, agg_sh.at[idx_d.at[_CPB - 1]], add=True)
            return 0
        lax.fori_loop(0, _NB, blk, 0)

        plsc.subcore_barrier()
        pltpu.sync_copy(agg_sh.at[pl.ds(s * _NPS, _NPS)], out_hbm.at[c, s])

    return pl.kernel(body, out_type=out_type, mesh=mesh,
                     scratch_types=scratch)


def _make_sc_cnt():
    mesh = plsc.VectorSubcoreMesh(
        core_axis_name="c", subcore_axis_name="s",
        num_cores=_NC, num_subcores=_NS)
    out_type = jax.ShapeDtypeStruct((_NC, _NS, _CNPS, _D), jnp.float32)
    scratch = [
        pltpu.VMEM((_CCPB, _CCH), jnp.int32),
        pltpu.VMEM((_CCH, _D), jnp.float32),
        pltpu.VMEM_SHARED((_N, _D), jnp.float32),
    ]

    def body(dst_hbm, zd_hbm, ones_hbm, *refs):
        (out_hbm, idx_d, ones_v, cnt_sh) = refs
        c = lax.axis_index("c")
        s = lax.axis_index("s")
        w = c * _NS + s

        pltpu.sync_copy(zd_hbm, cnt_sh.at[pl.ds(s * _CNPS, _CNPS)])
        pltpu.sync_copy(ones_hbm, ones_v)
        plsc.subcore_barrier()

        def blk(b, _):
            pltpu.sync_copy(dst_hbm.at[w, b], idx_d)

            def step(j, _):
                pltpu.sync_copy(ones_v, cnt_sh.at[idx_d.at[j]], add=True)
                return 0
            lax.fori_loop(0, _CCPB, step, 0)
            return 0
        lax.fori_loop(0, _CNB, blk, 0)

        plsc.subcore_barrier()
        pltpu.sync_copy(cnt_sh.at[pl.ds(s * _CNPS, _CNPS)], out_hbm.at[c, s])

    return pl.kernel(body, out_type=out_type, mesh=mesh,
                     scratch_types=scratch)


@functools.lru_cache(maxsize=None)
def _sc_agg_fn():
    return _make_sc_agg()


@functools.lru_cache(maxsize=None)
def _sc_cnt_fn():
    return _make_sc_cnt()


def _mlp_body(h_ref, p_ref, c_ref, pool_ref, w1_ref, b1_ref, g_ref, bt_ref,
              w2_ref, b2_ref, ho_ref, po_ref):
    cnt = c_ref[0, :, 0:1] + c_ref[1, :, 0:1]
    agg = (p_ref[0, :_N] + p_ref[1, :_N]) / jnp.maximum(cnt, 1.0)
    z = h_ref[...] + agg
    t = jnp.dot(z, w1_ref[...], preferred_element_type=jnp.float32) + b1_ref[...]
    mu = jnp.mean(t, axis=0, keepdims=True)
    d = t - mu
    var = jnp.mean(d * d, axis=0, keepdims=True)
    t = d * lax.rsqrt(var + _EPS) * g_ref[...] + bt_ref[...]
    t = jnp.maximum(t, 0.0)
    h = jnp.dot(t, w2_ref[...], preferred_element_type=jnp.float32) + b2_ref[...]
    ho_ref[...] = h
    po_ref[...] = pool_ref[...] + h


_tc_mlp = pl.pallas_call(
    _mlp_body,
    out_shape=(jax.ShapeDtypeStruct((_N, _D), jnp.float32),
               jax.ShapeDtypeStruct((_N, _D), jnp.float32)),
)


def _pool_body(pool_ref, b_ref, out_ref):
    oh = (b_ref[...] == lax.broadcasted_iota(jnp.int32, (1, _G), 1))
    oh = oh.astype(jnp.float32)
    cnts = jnp.sum(oh, axis=0, keepdims=True)
    ohn = oh / jnp.maximum(cnts, 1.0)
    out_ref[...] = lax.dot_general(
        ohn, pool_ref[...], (((0,), (0,)), ((), ())),
        preferred_element_type=jnp.float32)


_tc_pool = pl.pallas_call(
    _pool_body,
    out_shape=jax.ShapeDtypeStruct((_G, _D), jnp.float32),
)


def kernel(x, edge_index, batch, params):
    pad_src = jnp.broadcast_to((jnp.arange(_PAD, dtype=jnp.int32) * 41) % _N,
                               (_NW, _PAD))
    pad_dst = jnp.broadcast_to(_N + (jnp.arange(_PAD, dtype=jnp.int32) % _NT),
                               (_NW, _PAD))
    src3 = jnp.concatenate(
        [edge_index[0].reshape(_NW, _EPW), pad_src], axis=1
    ).reshape(_NW, _NB, _CPB, _CH)
    dst3 = jnp.concatenate(
        [edge_index[1].reshape(_NW, _EPW), pad_dst], axis=1
    ).reshape(_NW, _NB, _CPB, _CH)
    dst4 = edge_index[1].reshape(_NW, _CNB, _CCPB, _CCH)
    b2 = batch.reshape(_N, 1)

    zda = jnp.zeros((_NPS, _D), jnp.float32)
    zdc = jnp.zeros((_CNPS, _D), jnp.float32)
    ones = jnp.ones((_CCH, _D), jnp.float32)

    cntp = _sc_cnt_fn()(dst4, zdc, ones).reshape(_NC, _N, _D)
    aggp = _sc_agg_fn()(x, src3, dst3, zda).reshape(_NC, _NA, _D)
    pool = jnp.zeros((_N, _D), jnp.float32)
    h = x
    for l, (W1, b1, gm, bt, W2, b2_) in enumerate(params):
        if l > 0:
            aggp = _sc_agg_fn()(h, src3, dst3, zda).reshape(_NC, _NA, _D)
        h, pool = _tc_mlp(h, aggp, cntp, pool, W1, b1.reshape(1, _D),
                          gm.reshape(1, _D), bt.reshape(1, _D), W2,
                          b2_.reshape(1, _D))
    gpool = _tc_pool(pool, b2)
    return (pool, gpool)


# speedup vs baseline: 1.1799x; 1.1799x over previous
import functools

import jax
import jax.numpy as jnp
from jax import lax
from jax.experimental import pallas as pl
from jax.experimental.pallas import tpu as pltpu
from jax.experimental.pallas import tpu_sc as plsc

_N = 10000
_E = 320000
_D = 128
_G = 64
_EPS = 1e-5

_NC = 2
_NS = 16
_NW = _NC * _NS
_CH = 128
_EPW = _E // _NW
_PAD = 240
_EPWP = _EPW + _PAD
_RPW = _EPWP // _CH
_NB = 4
_CPB = _RPW // _NB
_NT = 16
_NA = _N + _NT
_NPS = _NA // _NS


def _make_sc_agg():
    mesh = plsc.VectorSubcoreMesh(
        core_axis_name="c", subcore_axis_name="s",
        num_cores=_NC, num_subcores=_NS)
    out_type = jax.ShapeDtypeStruct((_NC, _NS, _NPS, _D), jnp.float32)
    scratch = [
        pltpu.VMEM((_CPB, _CH), jnp.int32),
        pltpu.VMEM((_CPB, _CH), jnp.int32),
        pltpu.VMEM((_CH, _D), jnp.float32),
        pltpu.VMEM((_CH, _D), jnp.float32),
        pltpu.VMEM_SHARED((_NA, _D), jnp.float32),
        pltpu.SemaphoreType.DMA,
        pltpu.SemaphoreType.DMA,
        pltpu.SemaphoreType.DMA,
        pltpu.SemaphoreType.DMA,
    ]

    def body(h_hbm, src_hbm, dst_hbm, zd_hbm, *refs):
        (out_hbm, idx_s, idx_d, rows0, rows1, agg_sh,
         g0, g1, s0, s1) = refs
        rows = (rows0, rows1)
        gsem = (g0, g1)
        ssem = (s0, s1)
        c = lax.axis_index("c")
        s = lax.axis_index("s")
        w = c * _NS + s

        pltpu.sync_copy(zd_hbm, agg_sh.at[pl.ds(s * _NPS, _NPS)])

        plsc.subcore_barrier()

        def blk(b, _):
            pltpu.sync_copy(src_hbm.at[w, b], idx_s)
            pltpu.sync_copy(dst_hbm.at[w, b], idx_d)

            for j in range(_CPB):
                v = j & 1
                if j >= 2:
                    pltpu.make_async_copy(
                        rows[v], agg_sh.at[idx_d.at[j - 2]],
                        ssem[v]).wait()
                pltpu.async_copy(h_hbm.at[idx_s.at[j]], rows[v], gsem[v])
                if j >= 1:
                    pv = (j - 1) & 1
                    pltpu.make_async_copy(h_hbm.at[idx_s.at[j - 1]],
                                          rows[pv], gsem[pv]).wait()
                    pltpu.async_copy(rows[pv], agg_sh.at[idx_d.at[j - 1]],
                                     ssem[pv], add=True)
            lv = (_CPB - 1) & 1
            pltpu.make_async_copy(h_hbm.at[idx_s.at[_CPB - 1]], rows[lv],
                                  gsem[lv]).wait()
            pltpu.async_copy(rows[lv], agg_sh.at[idx_d.at[_CPB - 1]],
                             ssem[lv], add=True)
            pltpu.make_async_copy(rows[1 - lv],
                                  agg_sh.at[idx_d.at[_CPB - 2]],
                                  ssem[1 - lv]).wait()
            pltpu.make_async_copy(rows[lv], agg_sh.at[idx_d.at[_CPB - 1]],
                                  ssem[lv]).wait()
            return 0
        lax.fori_loop(0, _NB, blk, 0)

        plsc.subcore_barrier()
        pltpu.sync_copy(agg_sh.at[pl.ds(s * _NPS, _NPS)], out_hbm.at[c, s])

    return pl.kernel(body, out_type=out_type, mesh=mesh,
                     scratch_types=scratch)


def _make_sc_cnt():
    mesh = plsc.VectorSubcoreMesh(
        core_axis_name="c", subcore_axis_name="s",
        num_cores=_NC, num_subcores=_NS)
    out_type = jax.ShapeDtypeStruct((_NC, _NS, _NPS, _D), jnp.float32)
    scratch = [
        pltpu.VMEM((_CPB, _CH), jnp.int32),
        pltpu.VMEM((_CH, _D), jnp.float32),
        pltpu.VMEM_SHARED((_NA, _D), jnp.float32),
        pltpu.SemaphoreType.DMA,
        pltpu.SemaphoreType.DMA,
    ]

    def body(dst_hbm, zd_hbm, ones_hbm, *refs):
        (out_hbm, idx_d, ones_v, cnt_sh, s0, s1) = refs
        ssem = (s0, s1)
        c = lax.axis_index("c")
        s = lax.axis_index("s")
        w = c * _NS + s

        pltpu.sync_copy(zd_hbm, cnt_sh.at[pl.ds(s * _NPS, _NPS)])
        pltpu.sync_copy(ones_hbm, ones_v)
        plsc.subcore_barrier()

        def blk(b, _):
            pltpu.sync_copy(dst_hbm.at[w, b], idx_d)

            for j in range(_CPB):
                v = j & 1
                if j >= 2:
                    pltpu.make_async_copy(
                        ones_v, cnt_sh.at[idx_d.at[j - 2]], ssem[v]).wait()
                pltpu.async_copy(ones_v, cnt_sh.at[idx_d.at[j]], ssem[v],
                                 add=True)
            for j in range(_CPB - 2, _CPB):
                pltpu.make_async_copy(
                    ones_v, cnt_sh.at[idx_d.at[j]], ssem[j & 1]).wait()
            return 0
        lax.fori_loop(0, _NB, blk, 0)

        plsc.subcore_barrier()
        pltpu.sync_copy(cnt_sh.at[pl.ds(s * _NPS, _NPS)], out_hbm.at[c, s])

    return pl.kernel(body, out_type=out_type, mesh=mesh,
                     scratch_types=scratch)


@functools.lru_cache(maxsize=None)
def _sc_agg_fn():
    return _make_sc_agg()


@functools.lru_cache(maxsize=None)
def _sc_cnt_fn():
    return _make_sc_cnt()


def _mlp_body(h_ref, p_ref, c_ref, pool_ref, w1_ref, b1_ref, g_ref, bt_ref,
              w2_ref, b2_ref, ho_ref, po_ref):
    cnt = c_ref[0, :_N, 0:1] + c_ref[1, :_N, 0:1]
    agg = (p_ref[0, :_N] + p_ref[1, :_N]) / jnp.maximum(cnt, 1.0)
    z = h_ref[...] + agg
    t = jnp.dot(z, w1_ref[...], preferred_element_type=jnp.float32) + b1_ref[...]
    mu = jnp.mean(t, axis=0, keepdims=True)
    d = t - mu
    var = jnp.mean(d * d, axis=0, keepdims=True)
    t = d * lax.rsqrt(var + _EPS) * g_ref[...] + bt_ref[...]
    t = jnp.maximum(t, 0.0)
    h = jnp.dot(t, w2_ref[...], preferred_element_type=jnp.float32) + b2_ref[...]
    ho_ref[...] = h
    po_ref[...] = pool_ref[...] + h


_tc_mlp = pl.pallas_call(
    _mlp_body,
    out_shape=(jax.ShapeDtypeStruct((_N, _D), jnp.float32),
               jax.ShapeDtypeStruct((_N, _D), jnp.float32)),
)


def _pool_body(pool_ref, b_ref, out_ref):
    oh = (b_ref[...] == lax.broadcasted_iota(jnp.int32, (1, _G), 1))
    oh = oh.astype(jnp.float32)
    cnts = jnp.sum(oh, axis=0, keepdims=True)
    ohn = oh / jnp.maximum(cnts, 1.0)
    out_ref[...] = lax.dot_general(
        ohn, pool_ref[...], (((0,), (0,)), ((), ())),
        preferred_element_type=jnp.float32)


_tc_pool = pl.pallas_call(
    _pool_body,
    out_shape=jax.ShapeDtypeStruct((_G, _D), jnp.float32),
)


def kernel(x, edge_index, batch, params):
    pad_src = jnp.broadcast_to((jnp.arange(_PAD, dtype=jnp.int32) * 41) % _N,
                               (_NW, _PAD))
    pad_dst = jnp.broadcast_to(_N + (jnp.arange(_PAD, dtype=jnp.int32) % _NT),
                               (_NW, _PAD))
    src3 = jnp.concatenate(
        [edge_index[0].reshape(_NW, _EPW), pad_src], axis=1
    ).reshape(_NW, _NB, _CPB, _CH)
    dst3 = jnp.concatenate(
        [edge_index[1].reshape(_NW, _EPW), pad_dst], axis=1
    ).reshape(_NW, _NB, _CPB, _CH)
    b2 = batch.reshape(_N, 1)

    zda = jnp.zeros((_NPS, _D), jnp.float32)
    ones = jnp.ones((_CH, _D), jnp.float32)

    cntp = _sc_cnt_fn()(dst3, zda, ones).reshape(_NC, _NA, _D)
    aggp = _sc_agg_fn()(x, src3, dst3, zda).reshape(_NC, _NA, _D)
    pool = jnp.zeros((_N, _D), jnp.float32)
    h = x
    for l, (W1, b1, gm, bt, W2, b2_) in enumerate(params):
        if l > 0:
            aggp = _sc_agg_fn()(h, src3, dst3, zda).reshape(_NC, _NA, _D)
        h, pool = _tc_mlp(h, aggp, cntp, pool, W1, b1.reshape(1, _D),
                          gm.reshape(1, _D), bt.reshape(1, _D), W2,
                          b2_.reshape(1, _D))
    gpool = _tc_pool(pool, b2)
    return (pool, gpool)
